# baseline (speedup 1.0000x reference)
import functools

import jax
import jax.numpy as jnp
from jax.experimental import pallas as pl
from jax.experimental.pallas import tpu as pltpu

E = 8
TOP_K = 2
NEG_INF = float("-inf")


def _moe_tile_kernel(x_ref, wg_ref, we_hbm, be_ref, out_ref, w_vmem, sems):
    i = pl.program_id(0)

    @pl.when(i == 0)
    def _start_w_copies():
        for e in range(E):
            pltpu.make_async_copy(we_hbm.at[e], w_vmem.at[e], sems.at[e]).start()

    x = x_ref[...]
    tile = x.shape[0]

    logits = jax.lax.dot_general(
        x, wg_ref[...], (((1,), (1,)), ((), ())),
        preferred_element_type=jnp.float32)
    eids = jax.lax.broadcasted_iota(jnp.int32, (tile, E), 1)
    l1 = jnp.max(logits, axis=1, keepdims=True)
    i1 = jnp.min(jnp.where(logits == l1, eids, E), axis=1, keepdims=True)
    masked = jnp.where(eids == i1, NEG_INF, logits)
    l2 = jnp.max(masked, axis=1, keepdims=True)
    i2 = jnp.min(jnp.where(masked == l2, eids, E), axis=1, keepdims=True)
    e21 = jnp.exp(l2 - l1)
    w2 = e21 / (1.0 + e21)
    w1 = 1.0 - w2
    combine = jnp.where(eids == i1, w1, 0.0) + jnp.where(eids == i2, w2, 0.0)

    acc = jax.lax.dot_general(
        combine, be_ref[...], (((1,), (0,)), ((), ())),
        preferred_element_type=jnp.float32)
    for e in range(E):
        @pl.when(i == 0)
        def _wait_w(e=e):
            pltpu.make_async_copy(we_hbm.at[e], w_vmem.at[e], sems.at[e]).wait()

        y = jax.lax.dot_general(
            x, w_vmem[e], (((1,), (1,)), ((), ())),
            preferred_element_type=jnp.float32)
        acc += combine[:, e][:, None] * y
    out_ref[...] = acc


@functools.partial(jax.jit, static_argnames=())
def kernel(inputs, W_gate, W_experts, b_experts):
    batch_shape = inputs.shape[:-1]
    d_in = inputs.shape[-1]
    x = inputs.reshape(-1, d_in)
    t = x.shape[0]
    d_out = W_experts.shape[1]
    tile = 512
    grid = (t // tile,)

    out = pl.pallas_call(
        _moe_tile_kernel,
        grid=grid,
        in_specs=[
            pl.BlockSpec((tile, d_in), lambda i: (i, 0)),
            pl.BlockSpec((E, d_in), lambda i: (0, 0)),
            pl.BlockSpec(memory_space=pl.ANY),
            pl.BlockSpec((E, d_out), lambda i: (0, 0)),
        ],
        out_specs=pl.BlockSpec((tile, d_out), lambda i: (i, 0)),
        out_shape=jax.ShapeDtypeStruct((t, d_out), jnp.float32),
        scratch_shapes=[
            pltpu.VMEM((E, d_out, d_in), jnp.float32),
            pltpu.SemaphoreType.DMA((E,)),
        ],
    )(x, W_gate, W_experts, b_experts)
    return out.reshape(*batch_shape, d_out)


# speedup vs baseline: 1.5626x; 1.5626x over previous
import functools

import jax
import jax.numpy as jnp
from jax.experimental import pallas as pl
from jax.experimental.pallas import tpu as pltpu

E = 8
TOP_K = 2
NEG_INF = float("-inf")


def _moe_tile_kernel(x_ref, wg_ref, we_ref, be_ref, out_ref):
    x = x_ref[...]
    tile = x.shape[0]

    logits = jax.lax.dot_general(
        x, wg_ref[...], (((1,), (1,)), ((), ())),
        preferred_element_type=jnp.float32)
    eids = jax.lax.broadcasted_iota(jnp.int32, (tile, E), 1)
    l1 = jnp.max(logits, axis=1, keepdims=True)
    i1 = jnp.min(jnp.where(logits == l1, eids, E), axis=1, keepdims=True)
    masked = jnp.where(eids == i1, NEG_INF, logits)
    l2 = jnp.max(masked, axis=1, keepdims=True)
    i2 = jnp.min(jnp.where(masked == l2, eids, E), axis=1, keepdims=True)
    e21 = jnp.exp(l2 - l1)
    w2 = e21 / (1.0 + e21)
    w1 = 1.0 - w2
    combine = jnp.where(eids == i1, w1, 0.0) + jnp.where(eids == i2, w2, 0.0)

    acc = jax.lax.dot_general(
        combine, be_ref[...], (((1,), (0,)), ((), ())),
        preferred_element_type=jnp.float32)
    for e in range(E):
        y = jax.lax.dot_general(
            x, we_ref[e], (((1,), (1,)), ((), ())),
            preferred_element_type=jnp.float32)
        acc += combine[:, e][:, None] * y
    out_ref[...] = acc


@functools.partial(jax.jit, static_argnames=())
def kernel(inputs, W_gate, W_experts, b_experts):
    batch_shape = inputs.shape[:-1]
    d_in = inputs.shape[-1]
    x = inputs.reshape(-1, d_in)
    t = x.shape[0]
    d_out = W_experts.shape[1]
    tile = 512
    grid = (t // tile,)

    out = pl.pallas_call(
        _moe_tile_kernel,
        grid=grid,
        in_specs=[
            pl.BlockSpec((tile, d_in), lambda i: (i, 0)),
            pl.BlockSpec((E, d_in), lambda i: (0, 0)),
            pl.BlockSpec((E, d_out, d_in), lambda i: (0, 0, 0)),
            pl.BlockSpec((E, d_out), lambda i: (0, 0)),
        ],
        out_specs=pl.BlockSpec((tile, d_out), lambda i: (i, 0)),
        out_shape=jax.ShapeDtypeStruct((t, d_out), jnp.float32),
    )(x, W_gate, W_experts, b_experts)
    return out.reshape(*batch_shape, d_out)


# speedup vs baseline: 1.6424x; 1.0511x over previous
import functools

import jax
import jax.numpy as jnp
from jax.experimental import pallas as pl
from jax.experimental.pallas import tpu as pltpu

E = 8
TOP_K = 2
NEG_INF = float("-inf")


def _moe_tile_kernel(x_ref, wg_ref, we_ref, be_ref, out_ref):
    x = x_ref[...]
    tile = x.shape[0]

    logits = jax.lax.dot_general(
        x, wg_ref[...], (((1,), (1,)), ((), ())),
        preferred_element_type=jnp.float32)
    eids = jax.lax.broadcasted_iota(jnp.int32, (tile, E), 1)
    l1 = jnp.max(logits, axis=1, keepdims=True)
    i1 = jnp.min(jnp.where(logits == l1, eids, E), axis=1, keepdims=True)
    masked = jnp.where(eids == i1, NEG_INF, logits)
    l2 = jnp.max(masked, axis=1, keepdims=True)
    i2 = jnp.min(jnp.where(masked == l2, eids, E), axis=1, keepdims=True)
    e21 = jnp.exp(l2 - l1)
    w2 = e21 / (1.0 + e21)
    w1 = 1.0 - w2
    combine = jnp.where(eids == i1, w1, 0.0) + jnp.where(eids == i2, w2, 0.0)

    acc = jax.lax.dot_general(
        combine, be_ref[...], (((1,), (0,)), ((), ())),
        preferred_element_type=jnp.float32)
    for e in range(E):
        y = jax.lax.dot_general(
            x, we_ref[e], (((1,), (1,)), ((), ())),
            preferred_element_type=jnp.float32)
        acc += combine[:, e][:, None] * y
    out_ref[...] = acc


@functools.partial(jax.jit, static_argnames=())
def kernel(inputs, W_gate, W_experts, b_experts):
    batch_shape = inputs.shape[:-1]
    d_in = inputs.shape[-1]
    x = inputs.reshape(-1, d_in)
    t = x.shape[0]
    d_out = W_experts.shape[1]
    tile = 1024
    grid = (t // tile,)

    out = pl.pallas_call(
        _moe_tile_kernel,
        grid=grid,
        in_specs=[
            pl.BlockSpec((tile, d_in), lambda i: (i, 0)),
            pl.BlockSpec((E, d_in), lambda i: (0, 0)),
            pl.BlockSpec((E, d_out, d_in), lambda i: (0, 0, 0)),
            pl.BlockSpec((E, d_out), lambda i: (0, 0)),
        ],
        out_specs=pl.BlockSpec((tile, d_out), lambda i: (i, 0)),
        out_shape=jax.ShapeDtypeStruct((t, d_out), jnp.float32),
    )(x, W_gate, W_experts, b_experts)
    return out.reshape(*batch_shape, d_out)
